# baseline (speedup 1.0000x reference)
import jax
import jax.numpy as jnp
from jax.experimental import pallas as pl
from jax.experimental.pallas import tpu as pltpu

_ACT_DIM = 2048
_DICT = 32768
_K = 64
_B = 8

_BD = 1024
_NB = _DICT // _BD


def _sortable_key(z):
    bits = jax.lax.bitcast_convert_type(z, jnp.int32)
    return jnp.where(bits >= 0, bits, bits ^ jnp.int32(0x7FFFFFFF))


def _fused_kernel(x_ref, we_ref, be_ref, wd_ref, bd_ref, sz_ref, out_ref,
                  z_scr, key_scr, thr_scr, acc, acc2):
    p = pl.program_id(0)
    i = pl.program_id(1)

    @pl.when(p == 0)
    def _encode():
        zblk = jax.lax.dot_general(
            x_ref[...], we_ref[...], (((1,), (1,)), ((), ())),
            preferred_element_type=jnp.float32) + be_ref[...]
        z_scr[:, pl.ds(i * _BD, _BD)] = zblk
        key_scr[:, pl.ds(i * _BD, _BD)] = _sortable_key(zblk)

        @pl.when(i == _NB - 1)
        def _finish():
            key = key_scr[...]

            def body(_, carry):
                lo, hi = carry
                mid = (lo >> 1) + (hi >> 1) + (lo & hi & 1)
                cnt = jnp.sum((key >= mid).astype(jnp.int32), axis=1,
                              keepdims=True)
                ge = cnt >= _K
                return jnp.where(ge, mid, lo), jnp.where(ge, hi, mid)

            lo0 = jnp.full((_B, 1), jnp.iinfo(jnp.int32).min, jnp.int32)
            hi0 = jnp.full((_B, 1), jnp.iinfo(jnp.int32).max, jnp.int32)
            thr, _ = jax.lax.fori_loop(0, 32, body, (lo0, hi0))
            thr_scr[...] = jnp.broadcast_to(thr, (_B, 128))
            sz_ref[...] = jnp.where(key >= thr, z_scr[...], 0.0)

    @pl.when(p == 1)
    def _decode():
        @pl.when(i == 0)
        def _init():
            acc[...] = jnp.zeros_like(acc)
            acc2[...] = jnp.zeros_like(acc2)

        d = pl.ds(i * _BD, _BD)
        szblk = jnp.where(key_scr[:, d] >= thr_scr[:, :1], z_scr[:, d], 0.0)
        part = jax.lax.dot_general(
            szblk, wd_ref[...], (((1,), (1,)), ((), ())),
            preferred_element_type=jnp.float32)

        @pl.when(i % 2 == 0)
        def _even():
            acc[...] += part

        @pl.when(i % 2 == 1)
        def _odd():
            acc2[...] += part

        @pl.when(i == _NB - 1)
        def _finish():
            out_ref[...] = acc[...] + acc2[...] + bd_ref[...]


@jax.jit
def kernel(x, W_enc, b_enc, W_dec, b_dec):
    b_enc2 = b_enc.reshape(1, _DICT)
    b_dec2 = b_dec.reshape(1, _ACT_DIM)
    nb = _NB

    sparse_z, x_hat = pl.pallas_call(
        _fused_kernel,
        grid=(2, nb),
        in_specs=[
            pl.BlockSpec((_B, _ACT_DIM), lambda p, i: (0, 0)),
            pl.BlockSpec((_BD, _ACT_DIM),
                         lambda p, i: (jnp.where(p == 0, i, nb - 1), 0)),
            pl.BlockSpec((1, _BD),
                         lambda p, i: (0, jnp.where(p == 0, i, nb - 1))),
            pl.BlockSpec((_ACT_DIM, _BD),
                         lambda p, i: (0, jnp.where(p == 1, i, 0))),
            pl.BlockSpec((1, _ACT_DIM), lambda p, i: (0, 0)),
        ],
        out_specs=[
            pl.BlockSpec((_B, _DICT), lambda p, i: (0, 0)),
            pl.BlockSpec((_B, _ACT_DIM), lambda p, i: (0, 0)),
        ],
        out_shape=[
            jax.ShapeDtypeStruct((_B, _DICT), jnp.float32),
            jax.ShapeDtypeStruct((_B, _ACT_DIM), jnp.float32),
        ],
        scratch_shapes=[
            pltpu.VMEM((_B, _DICT), jnp.float32),
            pltpu.VMEM((_B, _DICT), jnp.int32),
            pltpu.VMEM((_B, 128), jnp.int32),
            pltpu.VMEM((_B, _ACT_DIM), jnp.float32),
            pltpu.VMEM((_B, _ACT_DIM), jnp.float32),
        ],
        compiler_params=pltpu.CompilerParams(
            dimension_semantics=("arbitrary", "arbitrary")),
    )(x, W_enc, b_enc2, W_dec, b_dec2)

    return (x_hat, sparse_z)
